# baseline (speedup 1.0000x reference)
import functools

import jax
import jax.numpy as jnp
from jax import lax
from jax.experimental import pallas as pl
from jax.experimental.pallas import tpu as pltpu
from jax.experimental.pallas import tpu_sc as plsc

B = 16384
D = 32
L = 16
NC = 2
NS = 16
NW = NC * NS
BPW = B // NW
CHUNK = 128
NCHUNK = BPW // CHUNK

LN2 = 0.6931471805599453
LN_MAXV = 13.815510557964274
SQRT2 = 1.4142135623730951
SCALE = 0.75

_mesh = plsc.VectorSubcoreMesh(core_axis_name="c", subcore_axis_name="s")


def _ln(x):
    bits = lax.bitcast_convert_type(x, jnp.int32)
    e = lax.shift_right_logical(bits, 23) - 127
    m = lax.bitcast_convert_type(
        (bits & 0x007FFFFF) | 0x3F800000, jnp.float32)
    big = m > SQRT2
    m = jnp.where(big, m * 0.5, m)
    e = e + jnp.where(big, 1, 0)
    z = (m - 1.0) / (m + 1.0)
    z2 = z * z
    p = z * (2.0 + z2 * (2.0 / 3.0 + z2 * (2.0 / 5.0 + z2 * (2.0 / 7.0 + z2 * (2.0 / 9.0)))))
    return e.astype(jnp.float32) * LN2 + p


@functools.partial(
    pl.kernel,
    out_type=jax.ShapeDtypeStruct((NW, L), jnp.float32),
    mesh=_mesh,
    compiler_params=pltpu.CompilerParams(needs_layout_passes=False,
                                         use_tc_tiling_on_sc=False),
    scratch_types=[
        pltpu.VMEM((NCHUNK, CHUNK), jnp.int32),
        pltpu.VMEM((NCHUNK, CHUNK), jnp.int32),
        pltpu.VMEM((BPW, D), jnp.float32),
        pltpu.VMEM((BPW, D), jnp.float32),
        pltpu.VMEM((BPW,), jnp.float32),
        pltpu.VMEM((BPW,), jnp.float32),
        pltpu.VMEM((BPW,), jnp.float32),
        pltpu.VMEM((L * (L + 1),), jnp.float32),
        pltpu.VMEM((L,), jnp.float32),
        pltpu.SemaphoreType.DMA,
    ],
)
def _glove_cost(t_ind, c_ind, co_hbm, t_emb, c_emb, t_bias, c_bias, out,
                t_idx, c_idx, t_rows, c_rows, tb_v, cb_v, co_v, pad, acc_v,
                sem):
    wid = lax.axis_index("s") * NC + lax.axis_index("c")
    base = wid * BPW

    for c in range(NCHUNK):
        pltpu.sync_copy(t_ind.at[pl.ds(base + c * CHUNK, CHUNK)], t_idx.at[c])
        pltpu.sync_copy(c_ind.at[pl.ds(base + c * CHUNK, CHUNK)], c_idx.at[c])
    pltpu.sync_copy(co_hbm.at[pl.ds(base, BPW)], co_v)

    copies = []
    for c in range(NCHUNK):
        sl = pl.ds(c * CHUNK, CHUNK)
        copies.append(pltpu.async_copy(t_emb.at[t_idx.at[c]], t_rows.at[sl], sem))
        copies.append(pltpu.async_copy(c_emb.at[c_idx.at[c]], c_rows.at[sl], sem))
        copies.append(pltpu.async_copy(t_bias.at[t_idx.at[c]], tb_v.at[sl], sem))
        copies.append(pltpu.async_copy(c_bias.at[c_idx.at[c]], cb_v.at[sl], sem))
    for cp in copies:
        cp.wait()

    lanes17 = lax.broadcasted_iota(jnp.int32, (L,), 0) * (L + 1)

    def group_body(g, acc):
        rbase = g * L
        for r in range(L):
            row = rbase + r
            a0 = t_rows[row, pl.ds(0, L)]
            a1 = t_rows[row, pl.ds(L, L)]
            b0 = c_rows[row, pl.ds(0, L)]
            b1 = c_rows[row, pl.ds(L, L)]
            pad[pl.ds(r * (L + 1), L)] = a0 * b0 + a1 * b1
        d = jnp.zeros((L,), jnp.float32)
        for j in range(L):
            d = d + plsc.load_gather(pad, [lanes17 + j])
        tb = tb_v[pl.ds(rbase, L)]
        cb = cb_v[pl.ds(rbase, L)]
        co = co_v[pl.ds(rbase, L)]
        lnco = _ln(co)
        w = jnp.minimum(1.0, jnp.exp(SCALE * (lnco - LN_MAXV)))
        err = d + tb + cb - lnco
        return acc + w * err * err

    acc = lax.fori_loop(0, BPW // L, group_body, jnp.zeros((L,), jnp.float32))
    acc_v[...] = acc
    pltpu.sync_copy(acc_v, out.at[wid])


def kernel(target_ind, context_ind, co_occurs, target_embeddings,
           context_embeddings, target_biases, context_biases):
    partials = _glove_cost(target_ind, context_ind, co_occurs,
                           target_embeddings, context_embeddings,
                           target_biases, context_biases)
    return jnp.sum(partials)
